# baseline (speedup 1.0000x reference)
import functools

import jax
import jax.numpy as jnp
from jax import lax
from jax.experimental import pallas as pl
from jax.experimental.pallas import tpu as pltpu
from jax.experimental.pallas import tpu_sc as plsc

N_IN = 50000
C = 128
M = 400000
N_OUT = 400000

NC = 2
NS = 16
NW = NC * NS

R = 320
RP = 336
NBLK = N_OUT // R
K = 128
BNDP = 1264
DR = 64
NPIECE = R // DR
CNT_OFF = RP * C
CNT_LEN = 352
ACC_W = CNT_OFF + CNT_LEN
ZTAIL = ACC_W - R * C


def _bcast(vec, lane_idx):
    return lax.gather(
        vec, lane_idx[:, None],
        lax.GatherDimensionNumbers(
            offset_dims=(), collapsed_slice_dims=(0,), start_index_map=(0,)),
        slice_sizes=(1,),
        mode=lax.GatherScatterMode.PROMISE_IN_BOUNDS)


def _body(feat_hbm, inmap_hbm, outmap_hbm, bnd_hbm, out_hbm,
          acc, rowbuf0, rowbuf1, zerobuf, zero_sh,
          inidx0, inidx1, outv0, outv1, locv0, locv1, bndv,
          sem_m, sem_g, sem_z, sem_wr):
    c = lax.axis_index("c")
    s = lax.axis_index("s")
    wid = s * NC + c
    pltpu.sync_copy(bnd_hbm, bndv)

    rowbufs = (rowbuf0, rowbuf1)
    inidxs = (inidx0, inidx1)
    outvs = (outv0, outv1)
    locvs = (locv0, locv1)

    def zrow(r, cz):
        zerobuf[pl.ds(16 * r, 16)] = jnp.zeros((16,), jnp.float32)
        return cz
    lax.fori_loop(0, DR * C // 16, zrow, 0)
    pltpu.sync_copy(zerobuf, zero_sh)
    plsc.subcore_barrier()

    def zero_issue(p):
        pltpu.async_copy(zero_sh, acc.at[pl.ds(p * DR * C, DR * C)], sem_z)

    def zero_issue_tail():
        pltpu.async_copy(zero_sh.at[pl.ds(0, ZTAIL)],
                         acc.at[pl.ds(R * C, ZTAIL)], sem_z)

    def wait_zeros():
        for p in range(NPIECE):
            pltpu.make_async_copy(
                zero_sh, acc.at[pl.ds(p * DR * C, DR * C)], sem_z).wait()
        pltpu.make_async_copy(
            zero_sh.at[pl.ds(0, ZTAIL)],
            acc.at[pl.ds(R * C, ZTAIL)], sem_z).wait()

    for p in range(NPIECE):
        zero_issue(p)
    zero_issue_tail()

    lanes = lax.broadcasted_iota(jnp.int32, (16,), 0)
    nb_w = (NBLK - wid + NW - 1) // NW

    def maps_issue(base, sb):
        win = jnp.minimum(base, M - K)
        pltpu.async_copy(inmap_hbm.at[pl.ds(win, K)], inidxs[sb], sem_m)
        pltpu.async_copy(outmap_hbm.at[pl.ds(win, K)], outvs[sb], sem_m)

    def maps_wait(sb):
        pltpu.make_async_copy(inmap_hbm.at[pl.ds(0, K)], inidxs[sb],
                              sem_m).wait()
        pltpu.make_async_copy(outmap_hbm.at[pl.ds(0, K)], outvs[sb],
                              sem_m).wait()

    def gather_issue(sb):
        pltpu.async_copy(feat_hbm.at[inidxs[sb]], rowbufs[sb], sem_g)

    def gather_wait(sb):
        pltpu.make_async_copy(feat_hbm.at[inidxs[sb]], rowbufs[sb],
                              sem_g).wait()

    def vec_scatter(base, rb, sb):
        win = jnp.minimum(base, M - K)

        ones16 = jnp.ones((16,), jnp.float32)

        def pass1(t, cg):
            ov = outvs[sb][pl.ds(16 * t, 16)]
            loc = ov - rb
            ev = win + 16 * t + lanes
            bad = (ev < base) | (loc < 0) | (loc >= R)
            loc = jnp.where(bad, R, loc)
            plsc.addupdate_scatter(acc, [CNT_OFF + loc], ones16)
            locvs[sb][pl.ds(16 * t, 16)] = loc.astype(jnp.float32)
            return cg
        lax.fori_loop(0, K // 16, pass1, 0)

        def pass2(t, cg):
            locf = locvs[sb][pl.ds(16 * t, 16)]
            for e in range(16):
                le = plsc.cummax(
                    jnp.where(lanes == e, locf, -1.0))[15].astype(jnp.int32)
                fbase = le * C + lanes
                for kk in range(C // 16):
                    vals = rowbufs[sb][16 * t + e, pl.ds(16 * kk, 16)]
                    plsc.addupdate_scatter(acc, [fbase + 16 * kk], vals)
            return cg
        lax.fori_loop(0, K // 16, pass2, 0)

    def do_block(b, carry):
        g = wid + b * NW
        rb = g * R
        ga = (g // 8) * 8
        grp = bndv[pl.ds(ga, 16)].astype(jnp.float32)
        j = g - ga
        lo = plsc.cummax(jnp.where(lanes == j, grp, -1.0))[15].astype(jnp.int32)
        hi = plsc.cummax(jnp.where(lanes == j + 1, grp, -1.0))[15].astype(jnp.int32)
        lo_al = (lo // 8) * 8
        nch = jnp.maximum((hi - lo_al + (K - 1)) // K, 1)

        wait_zeros()

        win0 = jnp.minimum(lo_al, M - K)
        pltpu.sync_copy(inmap_hbm.at[pl.ds(win0, K)], inidx0)
        pltpu.sync_copy(outmap_hbm.at[pl.ds(win0, K)], outv0)
        gather_issue(0)
        pl.when(nch > 1)(lambda: maps_issue(lo_al + K, 1))

        def chunk_iter(ci, sb):
            def stage_next():
                maps_wait(1 - sb)
                gather_issue(1 - sb)
            pl.when(ci + 1 < nch)(stage_next)
            gather_wait(sb)
            pl.when(ci + 2 < nch)(
                lambda: maps_issue(lo_al + (ci + 2) * K, sb))
            vec_scatter(lo_al + ci * K, rb, sb)

        def chunk_pair(cp, cc):
            ci = 2 * cp
            pl.when(ci < nch)(lambda: chunk_iter(ci, 0))
            pl.when(ci + 1 < nch)(lambda: chunk_iter(ci + 1, 1))
            return cc
        lax.fori_loop(0, (nch + 1) // 2, chunk_pair, 0)

        for t in range(RP // 16):
            cg = acc[pl.ds(CNT_OFF + 16 * t, 16)]
            acc[pl.ds(CNT_OFF + 16 * t, 16)] = 1.0 / jnp.maximum(cg, 1.0)

        def write_issue(p):
            pltpu.async_copy(acc.at[pl.ds(p * DR * C, DR * C)],
                             out_hbm.at[pl.ds((rb + p * DR) * C, DR * C)],
                             sem_wr)

        def write_wait(p):
            pltpu.make_async_copy(acc.at[pl.ds(p * DR * C, DR * C)],
                                  out_hbm.at[pl.ds((rb + p * DR) * C, DR * C)],
                                  sem_wr).wait()

        for p in range(NPIECE):
            def drow(r, cd):
                rg = (r // 16) * 16
                cgrp = acc[pl.ds(CNT_OFF + p * DR + rg, 16)]
                rec = _bcast(cgrp, jnp.full((16,), r - rg, jnp.int32))
                rbase = (p * DR + r) * C
                for kk in range(C // 16):
                    acc[pl.ds(rbase + 16 * kk, 16)] = (
                        acc[pl.ds(rbase + 16 * kk, 16)] * rec)
                return cd
            lax.fori_loop(0, DR, drow, 0)
            write_issue(p)
            if p >= 1:
                write_wait(p - 1)
                zero_issue(p - 1)
        write_wait(NPIECE - 1)
        zero_issue(NPIECE - 1)
        zero_issue_tail()
        return carry

    lax.fori_loop(0, nb_w, do_block, 0)
    wait_zeros()


@jax.jit
def _unpool(in_feat, in_map, out_map, bnd):
    mesh = plsc.VectorSubcoreMesh(
        core_axis_name="c", subcore_axis_name="s",
        num_cores=NC, num_subcores=NS)
    f = functools.partial(
        pl.kernel,
        out_type=jax.ShapeDtypeStruct((N_OUT * C,), jnp.float32),
        mesh=mesh,
        scratch_types=[
            pltpu.VMEM((ACC_W,), jnp.float32),
            pltpu.VMEM((K, C), jnp.float32),
            pltpu.VMEM((K, C), jnp.float32),
            pltpu.VMEM((DR * C,), jnp.float32),
            pltpu.VMEM_SHARED((DR * C,), jnp.float32),
            pltpu.VMEM((K,), jnp.int32),
            pltpu.VMEM((K,), jnp.int32),
            pltpu.VMEM((K,), jnp.int32),
            pltpu.VMEM((K,), jnp.int32),
            pltpu.VMEM((K,), jnp.float32),
            pltpu.VMEM((K,), jnp.float32),
            pltpu.VMEM((BNDP,), jnp.int32),
            pltpu.SemaphoreType.DMA,
            pltpu.SemaphoreType.DMA,
            pltpu.SemaphoreType.DMA,
            pltpu.SemaphoreType.DMA,
        ],
        compiler_params=pltpu.CompilerParams(needs_layout_passes=False),
    )(_body)
    return f(in_feat, in_map, out_map, bnd)


def kernel(in_feat, in_map, out_map, num_out):
    del num_out
    out_map = out_map.astype(jnp.int32)
    bnd = jnp.searchsorted(
        out_map,
        jnp.arange(0, N_OUT + 1, R, dtype=jnp.int32),
        side="left").astype(jnp.int32)
    bnd = jnp.pad(bnd, (0, BNDP - (NBLK + 1)), constant_values=M)
    out = _unpool(in_feat, in_map.astype(jnp.int32), out_map, bnd)
    return out.reshape(N_OUT, C)


# speedup vs baseline: 1.2108x; 1.2108x over previous
import functools

import jax
import jax.numpy as jnp
from jax import lax
from jax.experimental import pallas as pl
from jax.experimental.pallas import tpu as pltpu
from jax.experimental.pallas import tpu_sc as plsc

N_IN = 50000
C = 128
M = 400000
N_OUT = 400000

NC = 2
NS = 16
NW = NC * NS

R = 320
RP = 336
NBLK = N_OUT // R
K = 128
BNDP = 1264
DR = 64
NPIECE = R // DR


def _body(feat_hbm, inmap_hbm, outmap_hbm, bnd_hbm, out_hbm,
          acc_sh, rowbuf0, rowbuf1, dbuf0, dbuf1, zerobuf, cnt_v,
          inidx0, inidx1, outv0, outv1, locv0, locv1, bndv,
          sem_m, sem_g, sem_s, sem_z, sem_cp, sem_wr):
    c = lax.axis_index("c")
    s = lax.axis_index("s")
    wid = s * NC + c
    abase = s * RP
    pltpu.sync_copy(bnd_hbm, bndv)

    rowbufs = (rowbuf0, rowbuf1)
    dbufs = (dbuf0, dbuf1)
    inidxs = (inidx0, inidx1)
    outvs = (outv0, outv1)
    locvs = (locv0, locv1)

    def zrow(r, cz):
        z = jnp.zeros((16,), jnp.float32)
        for kk in range(C // 16):
            zerobuf[r, pl.ds(16 * kk, 16)] = z
        return cz
    lax.fori_loop(0, DR, zrow, 0)

    def zero_issue(p):
        pltpu.async_copy(zerobuf, acc_sh.at[pl.ds(abase + p * DR, DR)],
                         sem_z)

    def zero_issue_tail():
        pltpu.async_copy(zerobuf.at[pl.ds(0, RP - R)],
                         acc_sh.at[pl.ds(abase + R, RP - R)], sem_z)

    def wait_zeros():
        for p in range(NPIECE):
            pltpu.make_async_copy(
                zerobuf, acc_sh.at[pl.ds(abase + p * DR, DR)], sem_z).wait()
        pltpu.make_async_copy(
            zerobuf.at[pl.ds(0, RP - R)],
            acc_sh.at[pl.ds(abase + R, RP - R)], sem_z).wait()

    for p in range(NPIECE):
        zero_issue(p)
    zero_issue_tail()

    lanes = lax.broadcasted_iota(jnp.int32, (16,), 0)
    nb_w = (NBLK - wid + NW - 1) // NW

    def maps_issue(base, sb):
        win = jnp.minimum(base, M - K)
        pltpu.async_copy(inmap_hbm.at[pl.ds(win, K)], inidxs[sb], sem_m)
        pltpu.async_copy(outmap_hbm.at[pl.ds(win, K)], outvs[sb], sem_m)

    def maps_wait(sb):
        pltpu.make_async_copy(inmap_hbm.at[pl.ds(0, K)], inidxs[sb],
                              sem_m).wait()
        pltpu.make_async_copy(outmap_hbm.at[pl.ds(0, K)], outvs[sb],
                              sem_m).wait()

    def compute_locv(base, rb, sb):
        win = jnp.minimum(base, M - K)
        for t in range(K // 16):
            ov = outvs[sb][pl.ds(16 * t, 16)]
            loc = ov - rb
            ev = win + 16 * t + lanes
            bad = (ev < base) | (loc < 0) | (loc >= R)
            loc = jnp.where(bad, R, loc)
            plsc.addupdate_scatter(
                cnt_v, [loc], jnp.where(bad, 0.0, 1.0))
            locvs[sb][pl.ds(16 * t, 16)] = loc + abase

    def gather_issue(sb):
        pltpu.async_copy(feat_hbm.at[inidxs[sb]], rowbufs[sb], sem_g)

    def gather_wait(sb):
        pltpu.make_async_copy(feat_hbm.at[inidxs[sb]], rowbufs[sb],
                              sem_g).wait()

    def scatter_issue(sb):
        pltpu.async_copy(rowbufs[sb], acc_sh.at[locvs[sb]], sem_s, add=True)

    def scatter_wait(sb):
        pltpu.make_async_copy(rowbufs[sb], acc_sh.at[locvs[sb]],
                              sem_s).wait()

    def do_block(b, carry):
        g = wid + b * NW
        rb = g * R
        ga = (g // 8) * 8
        grp = bndv[pl.ds(ga, 16)].astype(jnp.float32)
        j = g - ga
        lo = plsc.cummax(jnp.where(lanes == j, grp, -1.0))[15].astype(jnp.int32)
        hi = plsc.cummax(jnp.where(lanes == j + 1, grp, -1.0))[15].astype(jnp.int32)
        lo_al = (lo // 8) * 8
        nch = jnp.maximum((hi - lo_al + (K - 1)) // K, 1)

        wait_zeros()
        for t in range(RP // 16):
            cnt_v[pl.ds(16 * t, 16)] = jnp.zeros((16,), jnp.float32)

        win0 = jnp.minimum(lo_al, M - K)
        pltpu.sync_copy(inmap_hbm.at[pl.ds(win0, K)], inidx0)
        pltpu.sync_copy(outmap_hbm.at[pl.ds(win0, K)], outv0)
        compute_locv(lo_al, rb, 0)
        gather_issue(0)
        pl.when(nch > 1)(lambda: maps_issue(lo_al + K, 1))

        def chunk_iter(ci, sb):
            pl.when(ci >= 1)(lambda: scatter_wait(1 - sb))

            def stage_next():
                maps_wait(1 - sb)
                compute_locv(lo_al + (ci + 1) * K, rb, 1 - sb)
                gather_issue(1 - sb)
            pl.when(ci + 1 < nch)(stage_next)
            gather_wait(sb)
            scatter_issue(sb)
            pl.when(ci + 2 < nch)(
                lambda: maps_issue(lo_al + (ci + 2) * K, sb))

        def chunk_pair(cp, cc):
            ci = 2 * cp
            pl.when(ci < nch)(lambda: chunk_iter(ci, 0))
            pl.when(ci + 1 < nch)(lambda: chunk_iter(ci + 1, 1))
            return cc
        lax.fori_loop(0, (nch + 1) // 2, chunk_pair, 0)
        pl.when(nch % 2 == 1)(lambda: scatter_wait(0))
        pl.when(nch % 2 == 0)(lambda: scatter_wait(1))

        for t in range(RP // 16):
            cg = cnt_v[pl.ds(16 * t, 16)]
            cnt_v[pl.ds(16 * t, 16)] = 1.0 / jnp.maximum(cg, 1.0)

        def copy_issue(p):
            pltpu.async_copy(acc_sh.at[pl.ds(abase + p * DR, DR)],
                             dbufs[p % 2], sem_cp)

        def copy_wait(p):
            pltpu.make_async_copy(acc_sh.at[pl.ds(abase + p * DR, DR)],
                                  dbufs[p % 2], sem_cp).wait()

        def write_issue(p):
            pltpu.async_copy(dbufs[p % 2],
                             out_hbm.at[pl.ds(rb + p * DR, DR)], sem_wr)

        def write_wait(p):
            pltpu.make_async_copy(dbufs[p % 2],
                                  out_hbm.at[pl.ds(rb + p * DR, DR)],
                                  sem_wr).wait()

        copy_issue(0)
        for p in range(NPIECE):
            if p + 1 < NPIECE:
                if p >= 1:
                    write_wait(p - 1)
                copy_issue(p + 1)
            copy_wait(p)
            zero_issue(p)

            def drow(r, cd):
                rg = (r // 16) * 16
                cgrp = cnt_v[pl.ds(p * DR + rg, 16)]
                bidx = jnp.full((16,), r - rg, jnp.int32)
                rec = lax.gather(
                    cgrp, bidx[:, None],
                    lax.GatherDimensionNumbers(
                        offset_dims=(), collapsed_slice_dims=(0,),
                        start_index_map=(0,)),
                    slice_sizes=(1,),
                    mode=lax.GatherScatterMode.PROMISE_IN_BOUNDS)
                for kk in range(C // 16):
                    dbufs[p % 2][r, pl.ds(16 * kk, 16)] = (
                        dbufs[p % 2][r, pl.ds(16 * kk, 16)] * rec)
                return cd
            lax.fori_loop(0, DR, drow, 0)
            write_issue(p)
        zero_issue_tail()
        write_wait(NPIECE - 2)
        write_wait(NPIECE - 1)
        return carry

    lax.fori_loop(0, nb_w, do_block, 0)
    wait_zeros()


@jax.jit
def _unpool(in_feat, in_map, out_map, bnd):
    mesh = plsc.VectorSubcoreMesh(
        core_axis_name="c", subcore_axis_name="s",
        num_cores=NC, num_subcores=NS)
    f = functools.partial(
        pl.kernel,
        out_type=jax.ShapeDtypeStruct((N_OUT, C), jnp.float32),
        mesh=mesh,
        scratch_types=[
            pltpu.VMEM_SHARED((NS * RP, C), jnp.float32),
            pltpu.VMEM((K, C), jnp.float32),
            pltpu.VMEM((K, C), jnp.float32),
            pltpu.VMEM((DR, C), jnp.float32),
            pltpu.VMEM((DR, C), jnp.float32),
            pltpu.VMEM((DR, C), jnp.float32),
            pltpu.VMEM((RP,), jnp.float32),
            pltpu.VMEM((K,), jnp.int32),
            pltpu.VMEM((K,), jnp.int32),
            pltpu.VMEM((K,), jnp.int32),
            pltpu.VMEM((K,), jnp.int32),
            pltpu.VMEM((K,), jnp.int32),
            pltpu.VMEM((K,), jnp.int32),
            pltpu.VMEM((BNDP,), jnp.int32),
            pltpu.SemaphoreType.DMA,
            pltpu.SemaphoreType.DMA,
            pltpu.SemaphoreType.DMA,
            pltpu.SemaphoreType.DMA,
            pltpu.SemaphoreType.DMA,
            pltpu.SemaphoreType.DMA,
        ],
        compiler_params=pltpu.CompilerParams(needs_layout_passes=False),
    )(_body)
    return f(in_feat, in_map, out_map, bnd)


def kernel(in_feat, in_map, out_map, num_out):
    del num_out
    out_map = out_map.astype(jnp.int32)
    bnd = jnp.searchsorted(
        out_map,
        jnp.arange(0, N_OUT + 1, R, dtype=jnp.int32),
        side="left").astype(jnp.int32)
    bnd = jnp.pad(bnd, (0, BNDP - (NBLK + 1)), constant_values=M)
    return _unpool(in_feat, in_map.astype(jnp.int32), out_map, bnd)


# speedup vs baseline: 1.2817x; 1.0586x over previous
import functools

import jax
import jax.numpy as jnp
from jax import lax
from jax.experimental import pallas as pl
from jax.experimental.pallas import tpu as pltpu
from jax.experimental.pallas import tpu_sc as plsc

N_IN = 50000
C = 128
M = 400000
N_OUT = 400000

NC = 2
NS = 16
NW = NC * NS

R = 400
RP = 416
NBLK = N_OUT // R
K = 128
BNDP = 1016
DR = 80
NPIECE = R // DR


def _body(feat_hbm, inmap_hbm, outmap_hbm, bnd_hbm, out_hbm,
          acc_sh, rowbuf0, rowbuf1, dbuf0, dbuf1, zerobuf, cnt_v,
          inidx0, inidx1, outv0, outv1, locv0, locv1, bndv,
          sem_m, sem_g, sem_s, sem_z, sem_cp, sem_wr):
    c = lax.axis_index("c")
    s = lax.axis_index("s")
    wid = s * NC + c
    abase = s * RP
    pltpu.sync_copy(bnd_hbm, bndv)

    rowbufs = (rowbuf0, rowbuf1)
    dbufs = (dbuf0, dbuf1)
    inidxs = (inidx0, inidx1)
    outvs = (outv0, outv1)
    locvs = (locv0, locv1)

    def zrow(r, cz):
        z = jnp.zeros((16,), jnp.float32)
        for kk in range(C // 16):
            zerobuf[r, pl.ds(16 * kk, 16)] = z
        return cz
    lax.fori_loop(0, DR, zrow, 0)

    def zero_issue(p):
        pltpu.async_copy(zerobuf, acc_sh.at[pl.ds(abase + p * DR, DR)],
                         sem_z)

    def zero_issue_tail():
        pltpu.async_copy(zerobuf.at[pl.ds(0, RP - R)],
                         acc_sh.at[pl.ds(abase + R, RP - R)], sem_z)

    def wait_zeros():
        for p in range(NPIECE):
            pltpu.make_async_copy(
                zerobuf, acc_sh.at[pl.ds(abase + p * DR, DR)], sem_z).wait()
        pltpu.make_async_copy(
            zerobuf.at[pl.ds(0, RP - R)],
            acc_sh.at[pl.ds(abase + R, RP - R)], sem_z).wait()

    for p in range(NPIECE):
        zero_issue(p)
    zero_issue_tail()

    lanes = lax.broadcasted_iota(jnp.int32, (16,), 0)
    nb_w = (NBLK - wid + NW - 1) // NW

    def maps_issue(base, sb):
        win = jnp.minimum(base, M - K)
        pltpu.async_copy(inmap_hbm.at[pl.ds(win, K)], inidxs[sb], sem_m)
        pltpu.async_copy(outmap_hbm.at[pl.ds(win, K)], outvs[sb], sem_m)

    def maps_wait(sb):
        pltpu.make_async_copy(inmap_hbm.at[pl.ds(0, K)], inidxs[sb],
                              sem_m).wait()
        pltpu.make_async_copy(outmap_hbm.at[pl.ds(0, K)], outvs[sb],
                              sem_m).wait()

    def compute_locv(base, rb, sb):
        win = jnp.minimum(base, M - K)
        for t in range(K // 16):
            ov = outvs[sb][pl.ds(16 * t, 16)]
            loc = ov - rb
            ev = win + 16 * t + lanes
            bad = (ev < base) | (loc < 0) | (loc >= R)
            loc = jnp.where(bad, R, loc)
            plsc.addupdate_scatter(
                cnt_v, [loc], jnp.where(bad, 0.0, 1.0))
            locvs[sb][pl.ds(16 * t, 16)] = loc + abase

    def gather_issue(sb):
        pltpu.async_copy(feat_hbm.at[inidxs[sb]], rowbufs[sb], sem_g)

    def gather_wait(sb):
        pltpu.make_async_copy(feat_hbm.at[inidxs[sb]], rowbufs[sb],
                              sem_g).wait()

    def scatter_issue(sb):
        pltpu.async_copy(rowbufs[sb], acc_sh.at[locvs[sb]], sem_s, add=True)

    def scatter_wait(sb):
        pltpu.make_async_copy(rowbufs[sb], acc_sh.at[locvs[sb]],
                              sem_s).wait()

    def do_block(b, carry):
        g = wid + b * NW
        rb = g * R
        ga = (g // 8) * 8
        grp = bndv[pl.ds(ga, 16)].astype(jnp.float32)
        j = g - ga
        lo = plsc.cummax(jnp.where(lanes == j, grp, -1.0))[15].astype(jnp.int32)
        hi = plsc.cummax(jnp.where(lanes == j + 1, grp, -1.0))[15].astype(jnp.int32)
        lo_al = (lo // 8) * 8
        nch = jnp.maximum((hi - lo_al + (K - 1)) // K, 1)

        wait_zeros()
        for t in range(RP // 16):
            cnt_v[pl.ds(16 * t, 16)] = jnp.zeros((16,), jnp.float32)

        win0 = jnp.minimum(lo_al, M - K)
        pltpu.sync_copy(inmap_hbm.at[pl.ds(win0, K)], inidx0)
        pltpu.sync_copy(outmap_hbm.at[pl.ds(win0, K)], outv0)
        compute_locv(lo_al, rb, 0)
        gather_issue(0)
        pl.when(nch > 1)(lambda: maps_issue(lo_al + K, 1))

        def chunk_iter(ci, sb):
            pl.when(ci >= 1)(lambda: scatter_wait(1 - sb))

            def stage_next():
                maps_wait(1 - sb)
                compute_locv(lo_al + (ci + 1) * K, rb, 1 - sb)
                gather_issue(1 - sb)
            pl.when(ci + 1 < nch)(stage_next)
            gather_wait(sb)
            scatter_issue(sb)
            pl.when(ci + 2 < nch)(
                lambda: maps_issue(lo_al + (ci + 2) * K, sb))

        def chunk_pair(cp, cc):
            ci = 2 * cp
            pl.when(ci < nch)(lambda: chunk_iter(ci, 0))
            pl.when(ci + 1 < nch)(lambda: chunk_iter(ci + 1, 1))
            return cc
        lax.fori_loop(0, (nch + 1) // 2, chunk_pair, 0)
        pl.when(nch % 2 == 1)(lambda: scatter_wait(0))
        pl.when(nch % 2 == 0)(lambda: scatter_wait(1))

        for t in range(RP // 16):
            cg = cnt_v[pl.ds(16 * t, 16)]
            cnt_v[pl.ds(16 * t, 16)] = 1.0 / jnp.maximum(cg, 1.0)

        def copy_issue(p):
            pltpu.async_copy(acc_sh.at[pl.ds(abase + p * DR, DR)],
                             dbufs[p % 2], sem_cp)

        def copy_wait(p):
            pltpu.make_async_copy(acc_sh.at[pl.ds(abase + p * DR, DR)],
                                  dbufs[p % 2], sem_cp).wait()

        def write_issue(p):
            pltpu.async_copy(dbufs[p % 2],
                             out_hbm.at[pl.ds(rb + p * DR, DR)], sem_wr)

        def write_wait(p):
            pltpu.make_async_copy(dbufs[p % 2],
                                  out_hbm.at[pl.ds(rb + p * DR, DR)],
                                  sem_wr).wait()

        copy_issue(0)
        for p in range(NPIECE):
            if p + 1 < NPIECE:
                if p >= 1:
                    write_wait(p - 1)
                copy_issue(p + 1)
            copy_wait(p)
            zero_issue(p)

            def drow(r, cd):
                rg = (r // 16) * 16
                cgrp = cnt_v[pl.ds(p * DR + rg, 16)]
                bidx = jnp.full((16,), r - rg, jnp.int32)
                rec = lax.gather(
                    cgrp, bidx[:, None],
                    lax.GatherDimensionNumbers(
                        offset_dims=(), collapsed_slice_dims=(0,),
                        start_index_map=(0,)),
                    slice_sizes=(1,),
                    mode=lax.GatherScatterMode.PROMISE_IN_BOUNDS)
                for kk in range(C // 16):
                    dbufs[p % 2][r, pl.ds(16 * kk, 16)] = (
                        dbufs[p % 2][r, pl.ds(16 * kk, 16)] * rec)
                return cd
            lax.fori_loop(0, DR, drow, 0)
            write_issue(p)
        zero_issue_tail()
        write_wait(NPIECE - 2)
        write_wait(NPIECE - 1)
        return carry

    lax.fori_loop(0, nb_w, do_block, 0)
    wait_zeros()


@jax.jit
def _unpool(in_feat, in_map, out_map, bnd):
    mesh = plsc.VectorSubcoreMesh(
        core_axis_name="c", subcore_axis_name="s",
        num_cores=NC, num_subcores=NS)
    f = functools.partial(
        pl.kernel,
        out_type=jax.ShapeDtypeStruct((N_OUT, C), jnp.float32),
        mesh=mesh,
        scratch_types=[
            pltpu.VMEM_SHARED((NS * RP, C), jnp.float32),
            pltpu.VMEM((K, C), jnp.float32),
            pltpu.VMEM((K, C), jnp.float32),
            pltpu.VMEM((DR, C), jnp.float32),
            pltpu.VMEM((DR, C), jnp.float32),
            pltpu.VMEM((DR, C), jnp.float32),
            pltpu.VMEM((RP,), jnp.float32),
            pltpu.VMEM((K,), jnp.int32),
            pltpu.VMEM((K,), jnp.int32),
            pltpu.VMEM((K,), jnp.int32),
            pltpu.VMEM((K,), jnp.int32),
            pltpu.VMEM((K,), jnp.int32),
            pltpu.VMEM((K,), jnp.int32),
            pltpu.VMEM((BNDP,), jnp.int32),
            pltpu.SemaphoreType.DMA,
            pltpu.SemaphoreType.DMA,
            pltpu.SemaphoreType.DMA,
            pltpu.SemaphoreType.DMA,
            pltpu.SemaphoreType.DMA,
            pltpu.SemaphoreType.DMA,
        ],
        compiler_params=pltpu.CompilerParams(needs_layout_passes=False),
    )(_body)
    return f(in_feat, in_map, out_map, bnd)


def kernel(in_feat, in_map, out_map, num_out):
    del num_out
    out_map = out_map.astype(jnp.int32)
    bnd = jnp.searchsorted(
        out_map,
        jnp.arange(0, N_OUT + 1, R, dtype=jnp.int32),
        side="left").astype(jnp.int32)
    bnd = jnp.pad(bnd, (0, BNDP - (NBLK + 1)), constant_values=M)
    return _unpool(in_feat, in_map.astype(jnp.int32), out_map, bnd)
